# baseline (speedup 1.0000x reference)
import functools

import jax
import jax.numpy as jnp
from jax import lax
from jax.experimental import pallas as pl
from jax.experimental.pallas import tpu as pltpu
from jax.experimental.pallas import tpu_sc as plsc

N_EMB = 1000000
D = 64
B = 4096
L = 200

_info = plsc.get_sparse_core_info()
NC = _info.num_cores
NS = _info.num_subcores
NW = NC * NS
CHUNKS_PER_W = B // NW
NBUF = 4
VPR = D // 16


def _emb_body(x_hbm, tok_hbm, pos_hbm, out_hbm, idx_v, pos_v, buf_v, *sems):
    gsems = sems[:NBUF]
    osems = sems[NBUF:]
    wid = lax.axis_index("s") * NC + lax.axis_index("c")
    base_chunk = wid * CHUNKS_PER_W

    pltpu.sync_copy(pos_hbm.at[pl.ds(0, L)], pos_v)
    idx_off = pl.multiple_of(base_chunk * L, 8)
    pltpu.sync_copy(x_hbm.at[pl.ds(idx_off, CHUNKS_PER_W * L)], idx_v)

    def gather_descs(c, b):
        off = pl.multiple_of(c * L, 8)
        d1 = pltpu.make_async_copy(
            tok_hbm.at[idx_v.at[pl.ds(off, 128)]],
            buf_v.at[b, pl.ds(0, 128)],
            gsems[b])
        d2 = pltpu.make_async_copy(
            tok_hbm.at[idx_v.at[pl.ds(off + 128, L - 128)]],
            buf_v.at[b, pl.ds(128, L - 128)],
            gsems[b])
        return d1, d2

    def out_desc(c, b):
        goff = (base_chunk + c) * L
        return pltpu.make_async_copy(
            buf_v.at[b], out_hbm.at[pl.ds(goff, L)], osems[b])

    def add_pos(b):
        def row(r, carry):
            for d in range(VPR):
                sl = pl.ds(16 * d, 16)
                plsc.addupdate(buf_v.at[b, r, sl], pos_v[r, sl])
            return carry
        lax.fori_loop(0, L, row, 0)

    def chunk_step(c, b, pf_c=None, pf_b=0, pf_wait=True):
        d1, d2 = gather_descs(c, b)
        d1.wait()
        d2.wait()
        add_pos(b)
        out_desc(c, b).start()
        if pf_c is not None:
            if pf_wait:
                out_desc(c, pf_b).wait()
            g1, g2 = gather_descs(pf_c, pf_b)
            g1.start()
            g2.start()

    for c in (0, 1):
        d1, d2 = gather_descs(c, c)
        d1.start()
        d2.start()
    chunk_step(0, 0, pf_c=2, pf_b=2, pf_wait=False)
    chunk_step(1, 1, pf_c=3, pf_b=3, pf_wait=False)

    def group(g, carry):
        for j in range(4):
            c = 2 + g * 4 + j
            b = (2 + j) % NBUF
            chunk_step(c, b, pf_c=c + 2, pf_b=(b + 2) % NBUF, pf_wait=True)
        return carry
    lax.fori_loop(0, (CHUNKS_PER_W - 4) // 4, group, 0)

    chunk_step(CHUNKS_PER_W - 2, 2)
    chunk_step(CHUNKS_PER_W - 1, 3)

    for b in range(NBUF):
        out_desc(CHUNKS_PER_W - NBUF + b, b).wait()


_emb_call = functools.partial(
    pl.kernel,
    mesh=plsc.VectorSubcoreMesh(core_axis_name="c", subcore_axis_name="s"),
    out_type=jax.ShapeDtypeStruct((B * L, D), jnp.float32),
    scratch_types=[
        pltpu.VMEM((CHUNKS_PER_W * L,), jnp.int32),
        pltpu.VMEM((L, D), jnp.float32),
        pltpu.VMEM((NBUF, L, D), jnp.float32),
    ] + [pltpu.SemaphoreType.DMA] * (2 * NBUF),
    compiler_params=pltpu.CompilerParams(use_tc_tiling_on_sc=False),
)(_emb_body)


def kernel(x, token_table, pos_table):
    x_flat = x.reshape(-1).astype(jnp.int32)
    out = _emb_call(x_flat, token_table, pos_table)
    return out.reshape(B, L, D)
